# baseline (speedup 1.0000x reference)
import functools

import jax
import jax.numpy as jnp
from jax import lax
from jax.experimental import pallas as pl
from jax.experimental.pallas import tpu as pltpu
from jax.experimental.pallas import tpu_sc as plsc

N = 10000
E = 160000
DN = 128
DE = 16
DL = 8
DH = 16
DHID = 4096
MP = 4
PS = 3

NC = 2
NS = 16
NW = NC * NS
EPW = E // NW
CH = 125
NCHUNK = EPW // CH
NPW = N // NS

@functools.cache
def _sc_mesh():
    return plsc.VectorSubcoreMesh(
        core_axis_name="c", subcore_axis_name="s", num_cores=NC, num_subcores=NS
    )


def _gather_body(x_hbm, src_hbm, out_hbm, idx_v, rows_v, sem):
    wid = lax.axis_index("s") * NC + lax.axis_index("c")
    base = wid * EPW
    pltpu.sync_copy(src_hbm.at[pl.ds(base, EPW)], idx_v)
    pltpu.async_copy(x_hbm.at[idx_v], rows_v, sem).wait()
    pltpu.sync_copy(rows_v, out_hbm.at[pl.ds(base, EPW)])


def _sc_gather(x, src):
    return pl.kernel(
        _gather_body,
        out_type=jax.ShapeDtypeStruct((E, DH), jnp.float32),
        mesh=_sc_mesh(),
        compiler_params=pltpu.CompilerParams(use_tc_tiling_on_sc=False),
        scratch_types=[
            pltpu.VMEM((EPW,), jnp.int32),
            pltpu.VMEM((EPW, DH), jnp.float32),
            pltpu.SemaphoreType.DMA,
        ],
    )(x, src)


def _scatter_body(m_hbm, dst_hbm, zeros_hbm, out_hbm, idx_v, rows_v, acc_sh, sem):
    c = lax.axis_index("c")
    s = lax.axis_index("s")
    wid = s * NC + c
    pltpu.sync_copy(zeros_hbm.at[pl.ds(s * NPW, NPW)],
                    acc_sh.at[pl.ds(s * NPW, NPW)])
    plsc.subcore_barrier()
    pltpu.sync_copy(m_hbm.at[pl.ds(wid * EPW, EPW)], rows_v)
    pltpu.sync_copy(dst_hbm.at[wid], idx_v)

    def chunk(j, carry):
        pltpu.sync_copy(rows_v.at[pl.ds(j * CH, CH)],
                        acc_sh.at[idx_v.at[j]], add=True)
        return carry

    lax.fori_loop(0, NCHUNK, chunk, 0)
    plsc.subcore_barrier()
    pltpu.sync_copy(acc_sh.at[pl.ds(s * NPW, NPW)],
                    out_hbm.at[c, pl.ds(s * NPW, NPW)])


def _sc_scatter(m, dst3, zeros_n):
    return pl.kernel(
        _scatter_body,
        out_type=jax.ShapeDtypeStruct((NC, N, DH), jnp.float32),
        mesh=_sc_mesh(),
        compiler_params=pltpu.CompilerParams(use_tc_tiling_on_sc=False),
        scratch_types=[
            pltpu.VMEM((NCHUNK, CH), jnp.int32),
            pltpu.VMEM((EPW, DH), jnp.float32),
            pltpu.VMEM_SHARED((N, DH), jnp.float32),
            pltpu.SemaphoreType.DMA,
        ],
    )(m, dst3, zeros_n)


TN0 = 1000


def _nodeproj_body(na_ref, w1_ref, b1_ref, out_ref):
    out_ref[...] = jnp.maximum(
        jnp.dot(na_ref[...], w1_ref[...],
                preferred_element_type=jnp.float32) + b1_ref[...], 0.0)


def _tc_nodeproj(node_attribute, W1, b1):
    return pl.pallas_call(
        _nodeproj_body,
        grid=(N // TN0,),
        in_specs=[
            pl.BlockSpec((TN0, DN), lambda i: (i, 0)),
            pl.BlockSpec((DN, DH), lambda i: (0, 0)),
            pl.BlockSpec((1, DH), lambda i: (0, 0)),
        ],
        out_specs=pl.BlockSpec((TN0, DH), lambda i: (i, 0)),
        out_shape=jax.ShapeDtypeStruct((N, DH), jnp.float32),
    )(node_attribute, W1, b1.reshape(1, DH))


TEB = 640


import numpy as _np

_REP = _np.zeros((DH, DH * DH), _np.float32)
_FOLD = _np.zeros((DH * DH, DH), _np.float32)
for _i in range(DH):
    for _o in range(DH):
        _REP[_i, DH * _i + _o] = 1.0
        _FOLD[DH * _i + _o, _o] = 1.0


def _edge_body(xs_ref, ea_ref, el_ref, wba_ref, wbb_ref, bb_ref, cen_ref,
               beta_ref, rep_ref, fold_ref, m_ref):
    diff = el_ref[...] - cen_ref[...]

    rbf = jnp.exp(-beta_ref[...] * diff * diff)
    wf = jnp.dot(ea_ref[...], wba_ref[...],
                 preferred_element_type=jnp.float32) \
        + jnp.dot(rbf, wbb_ref[...],
                  preferred_element_type=jnp.float32) + bb_ref[...]
    xr = jnp.dot(xs_ref[...], rep_ref[...],
                 preferred_element_type=jnp.float32)
    m_ref[...] = jnp.dot(wf * xr, fold_ref[...],
                         preferred_element_type=jnp.float32)


def _tc_edge(xs, edge_attribute, el8, Wb, bb, centers, beta):
    return pl.pallas_call(
        _edge_body,
        grid=(E // TEB,),
        in_specs=[
            pl.BlockSpec((TEB, DH), lambda i: (i, 0)),
            pl.BlockSpec((TEB, DE), lambda i: (i, 0)),
            pl.BlockSpec((TEB, DL), lambda i: (i, 0)),
            pl.BlockSpec((DE, DH * DH), lambda i: (0, 0)),
            pl.BlockSpec((DL, DH * DH), lambda i: (0, 0)),
            pl.BlockSpec((1, DH * DH), lambda i: (0, 0)),
            pl.BlockSpec((1, DL), lambda i: (0, 0)),
            pl.BlockSpec((1, DL), lambda i: (0, 0)),
            pl.BlockSpec((DH, DH * DH), lambda i: (0, 0)),
            pl.BlockSpec((DH * DH, DH), lambda i: (0, 0)),
        ],
        out_specs=pl.BlockSpec((TEB, DH), lambda i: (i, 0)),
        out_shape=jax.ShapeDtypeStruct((E, DH), jnp.float32),
    )(xs, edge_attribute, el8, Wb[:DE], Wb[DE:], bb.reshape(1, DH * DH),
      centers.reshape(1, DL), beta.reshape(1, DL),
      jnp.asarray(_REP), jnp.asarray(_FOLD))


TNG = N


def _gru_body(p_ref, h_ref, bias_ref, wih_ref, whh_ref, bih_ref, bhh_ref,
              hn_ref):
    agg = p_ref[0] + p_ref[1] + bias_ref[...]
    x = jnp.maximum(agg, 0.0)
    gi = jnp.dot(x, wih_ref[...], preferred_element_type=jnp.float32) \
        + bih_ref[...]
    gh = jnp.dot(h_ref[...], whh_ref[...],
                 preferred_element_type=jnp.float32) + bhh_ref[...]
    r = jax.nn.sigmoid(gi[:, :DH] + gh[:, :DH])
    z = jax.nn.sigmoid(gi[:, DH:2 * DH] + gh[:, DH:2 * DH])
    n = jnp.tanh(gi[:, 2 * DH:] + r * gh[:, 2 * DH:])
    hn_ref[...] = (1.0 - z) * n + z * h_ref[...]


def _tc_gru(parts, h, gnn_bias, WihT, WhhT, bih, bhh):
    return pl.pallas_call(
        _gru_body,
        grid=(1,),
        in_specs=[
            pl.BlockSpec((NC, TNG, DH), lambda i: (0, 0, 0)),
            pl.BlockSpec((TNG, DH), lambda i: (0, 0)),
            pl.BlockSpec((1, DH), lambda i: (0, 0)),
            pl.BlockSpec((DH, 3 * DH), lambda i: (0, 0)),
            pl.BlockSpec((DH, 3 * DH), lambda i: (0, 0)),
            pl.BlockSpec((1, 3 * DH), lambda i: (0, 0)),
            pl.BlockSpec((1, 3 * DH), lambda i: (0, 0)),
        ],
        out_specs=pl.BlockSpec((TNG, DH), lambda i: (0, 0)),
        out_shape=jax.ShapeDtypeStruct((N, DH), jnp.float32),
    )(parts, h, gnn_bias.reshape(1, DH), WihT, WhhT,
      bih.reshape(1, 3 * DH), bhh.reshape(1, 3 * DH))


H2 = 2 * DH


def _s2s_body(x_ref, nf_ref, wih0_ref, whh0_ref, bih0_ref, bhh0_ref,
              wih1_ref, whh1_ref, bih1_ref, bhh1_ref, ws_ref, bs_ref,
              pa_ref, out_ref):
    x = x_ref[...]
    nf = nf_ref[...]
    h0 = jnp.zeros((1, H2), jnp.float32)
    c0 = jnp.zeros((1, H2), jnp.float32)
    h1 = jnp.zeros((1, H2), jnp.float32)
    c1 = jnp.zeros((1, H2), jnp.float32)
    q_star = jnp.zeros((1, 2 * H2), jnp.float32)

    def cell(xx, hh, cc, wih, whh, bih, bhh):
        g = jnp.dot(xx, wih, preferred_element_type=jnp.float32) + bih \
            + jnp.dot(hh, whh, preferred_element_type=jnp.float32) + bhh
        i = jax.nn.sigmoid(g[:, :H2])
        f = jax.nn.sigmoid(g[:, H2:2 * H2])
        gg = jnp.tanh(g[:, 2 * H2:3 * H2])
        o = jax.nn.sigmoid(g[:, 3 * H2:])
        cc2 = f * cc + i * gg
        return o * jnp.tanh(cc2), cc2

    for _ in range(PS):
        h0, c0 = cell(q_star, h0, c0, wih0_ref[...], whh0_ref[...],
                      bih0_ref[...], bhh0_ref[...])
        h1, c1 = cell(h0, h1, c1, wih1_ref[...], whh1_ref[...],
                      bih1_ref[...], bhh1_ref[...])
        q = h1
        e = lax.dot_general(x, q[:, :DH], (((1,), (1,)), ((), ())),
                            preferred_element_type=jnp.float32) \
            + lax.dot_general(nf, q[:, DH:], (((1,), (1,)), ((), ())),
                              preferred_element_type=jnp.float32)
        p = jnp.exp(e - jnp.max(e))
        alpha = p / jnp.sum(p)
        r1 = lax.dot_general(alpha, x, (((0,), (0,)), ((), ())),
                             preferred_element_type=jnp.float32)
        r2 = lax.dot_general(alpha, nf, (((0,), (0,)), ((), ())),
                             preferred_element_type=jnp.float32)
        q_star = jnp.concatenate([q, r1, r2], axis=1)

    out = jnp.dot(q_star, ws_ref[...],
                  preferred_element_type=jnp.float32) + bs_ref[...]
    out_ref[...] = jnp.maximum(out, 0.0) + pa_ref[0, 0] * jnp.minimum(out, 0.0)


def _tc_s2s(x, nf, WihT0, WhhT0, bih0, bhh0, WihT1, WhhT1, bih1, bhh1,
            Ws, bs, prelu_a):
    full = lambda shape: pl.BlockSpec(shape, lambda: tuple(0 for _ in shape))
    return pl.pallas_call(
        _s2s_body,
        grid=(),
        in_specs=[
            full((N, DH)), full((N, DH)),
            full((2 * H2, 4 * H2)), full((H2, 4 * H2)),
            full((1, 4 * H2)), full((1, 4 * H2)),
            full((H2, 4 * H2)), full((H2, 4 * H2)),
            full((1, 4 * H2)), full((1, 4 * H2)),
            full((4 * DH, DHID)), full((1, DHID)), full((1, 1)),
        ],
        out_specs=full((1, DHID)),
        out_shape=jax.ShapeDtypeStruct((1, DHID), jnp.float32),
    )(x, nf, WihT0, WhhT0, bih0.reshape(1, -1), bhh0.reshape(1, -1),
      WihT1, WhhT1, bih1.reshape(1, -1), bhh1.reshape(1, -1),
      Ws, bs.reshape(1, DHID), prelu_a.reshape(1, 1))


def kernel(node_attribute, edge_index, edge_attribute, edge_length, W1, b1,
           rbf_centers, rbf_beta, Wb, bb, gnn_bias, gru_Wih, gru_Whh,
           gru_bih, gru_bhh, lstm_Wih0, lstm_Whh0, lstm_bih0, lstm_bhh0,
           lstm_Wih1, lstm_Whh1, lstm_bih1, lstm_bhh1, Ws, bs, prelu_a):
    src = edge_index[0]
    dst3 = edge_index[1].reshape(NW, NCHUNK, CH)
    el8 = jnp.broadcast_to(edge_length.reshape(E, 1), (E, DL))
    zeros_n = jnp.zeros((N, DH), jnp.float32)
    WihT = gru_Wih.T
    WhhT = gru_Whh.T
    WihT0 = lstm_Wih0.T
    WhhT0 = lstm_Whh0.T
    WihT1 = lstm_Wih1.T
    WhhT1 = lstm_Whh1.T

    nf = _tc_nodeproj(node_attribute, W1, b1)
    x = nf
    h = nf
    for _ in range(MP):
        xs = _sc_gather(x, src)
        m = _tc_edge(xs, edge_attribute, el8, Wb, bb, rbf_centers, rbf_beta)
        parts = _sc_scatter(m, dst3, zeros_n)
        h = _tc_gru(parts, h, gnn_bias, WihT, WhhT, gru_bih, gru_bhh)
        x = h

    return _tc_s2s(x, nf, WihT0, WhhT0, lstm_bih0, lstm_bhh0,
                   WihT1, WhhT1, lstm_bih1, lstm_bhh1, Ws, bs, prelu_a)


# speedup vs baseline: 1.3770x; 1.3770x over previous
import functools

import jax
import jax.numpy as jnp
from jax import lax
from jax.experimental import pallas as pl
from jax.experimental.pallas import tpu as pltpu
from jax.experimental.pallas import tpu_sc as plsc

N = 10000
E = 160000
DN = 128
DE = 16
DL = 8
DH = 16
DHID = 4096
MP = 4
PS = 3

NC = 2
NS = 16
NW = NC * NS
EPW = E // NW
CH = 125
NCHUNK = EPW // CH
NPW = N // NS

@functools.cache
def _sc_mesh():
    return plsc.VectorSubcoreMesh(
        core_axis_name="c", subcore_axis_name="s", num_cores=NC, num_subcores=NS
    )


def _gather_body(x_hbm, src_hbm, out_hbm, idx_v, rows_v, sem):
    wid = lax.axis_index("s") * NC + lax.axis_index("c")
    base = wid * EPW
    pltpu.sync_copy(src_hbm.at[pl.ds(base, EPW)], idx_v)
    pltpu.async_copy(x_hbm.at[idx_v], rows_v, sem).wait()
    pltpu.sync_copy(rows_v, out_hbm.at[pl.ds(base, EPW)])


def _sc_gather(x, src):
    return pl.kernel(
        _gather_body,
        out_type=jax.ShapeDtypeStruct((E, DH), jnp.float32),
        mesh=_sc_mesh(),
        compiler_params=pltpu.CompilerParams(use_tc_tiling_on_sc=False),
        scratch_types=[
            pltpu.VMEM((EPW,), jnp.int32),
            pltpu.VMEM((EPW, DH), jnp.float32),
            pltpu.SemaphoreType.DMA,
        ],
    )(x, src)


def _scatter_body(m_hbm, dst_hbm, zeros_hbm, out_hbm, idx_v, rows_v, acc_sh, sem):
    c = lax.axis_index("c")
    s = lax.axis_index("s")
    wid = s * NC + c
    pltpu.sync_copy(zeros_hbm.at[pl.ds(s * NPW, NPW)],
                    acc_sh.at[pl.ds(s * NPW, NPW)])
    plsc.subcore_barrier()
    pltpu.sync_copy(m_hbm.at[pl.ds(wid * EPW, EPW)], rows_v)
    pltpu.sync_copy(dst_hbm.at[wid], idx_v)

    def chunk(j, carry):
        pltpu.sync_copy(rows_v.at[pl.ds(j * CH, CH)],
                        acc_sh.at[idx_v.at[j]], add=True)
        return carry

    lax.fori_loop(0, NCHUNK, chunk, 0)
    plsc.subcore_barrier()
    pltpu.sync_copy(acc_sh.at[pl.ds(s * NPW, NPW)],
                    out_hbm.at[c, pl.ds(s * NPW, NPW)])


def _sc_scatter(m, dst3, zeros_n):
    return pl.kernel(
        _scatter_body,
        out_type=jax.ShapeDtypeStruct((NC, N, DH), jnp.float32),
        mesh=_sc_mesh(),
        compiler_params=pltpu.CompilerParams(use_tc_tiling_on_sc=False),
        scratch_types=[
            pltpu.VMEM((NCHUNK, CH), jnp.int32),
            pltpu.VMEM((EPW, DH), jnp.float32),
            pltpu.VMEM_SHARED((N, DH), jnp.float32),
            pltpu.SemaphoreType.DMA,
        ],
    )(m, dst3, zeros_n)


TN0 = 1000


def _nodeproj_body(na_ref, w1_ref, b1_ref, out_ref):
    out_ref[...] = jnp.maximum(
        jnp.dot(na_ref[...], w1_ref[...],
                preferred_element_type=jnp.float32) + b1_ref[...], 0.0)


def _tc_nodeproj(node_attribute, W1, b1):
    return pl.pallas_call(
        _nodeproj_body,
        grid=(N // TN0,),
        in_specs=[
            pl.BlockSpec((TN0, DN), lambda i: (i, 0)),
            pl.BlockSpec((DN, DH), lambda i: (0, 0)),
            pl.BlockSpec((1, DH), lambda i: (0, 0)),
        ],
        out_specs=pl.BlockSpec((TN0, DH), lambda i: (i, 0)),
        out_shape=jax.ShapeDtypeStruct((N, DH), jnp.float32),
    )(node_attribute, W1, b1.reshape(1, DH))


TEB = 3200


import numpy as _np

_REP = _np.zeros((DH, DH * DH), _np.float32)
_FOLD = _np.zeros((DH * DH, DH), _np.float32)
for _i in range(DH):
    for _o in range(DH):
        _REP[_i, DH * _i + _o] = 1.0
        _FOLD[DH * _i + _o, _o] = 1.0


SUB = 640
NSUB = TEB // SUB


def _edge_body(xs_ref, ea_ref, el_ref, wba_ref, wbb_ref, bb_ref, cen_ref,
               beta_ref, rep_ref, fold_ref, m_ref):
    for sub in range(NSUB):
        sl = pl.ds(sub * SUB, SUB)
        diff = el_ref[sl, :] - cen_ref[...]
        rbf = jnp.exp(-beta_ref[...] * diff * diff)
        wf = jnp.dot(ea_ref[sl, :], wba_ref[...],
                     preferred_element_type=jnp.float32) \
            + jnp.dot(rbf, wbb_ref[...],
                      preferred_element_type=jnp.float32) + bb_ref[...]
        xr = jnp.dot(xs_ref[sl, :], rep_ref[...],
                     preferred_element_type=jnp.float32)
        m_ref[sl, :] = jnp.dot(wf * xr, fold_ref[...],
                               preferred_element_type=jnp.float32)


def _tc_edge(xs, edge_attribute, el8, Wb, bb, centers, beta):
    return pl.pallas_call(
        _edge_body,
        grid=(E // TEB,),
        in_specs=[
            pl.BlockSpec((TEB, DH), lambda i: (i, 0)),
            pl.BlockSpec((TEB, DE), lambda i: (i, 0)),
            pl.BlockSpec((TEB, DL), lambda i: (i, 0)),
            pl.BlockSpec((DE, DH * DH), lambda i: (0, 0)),
            pl.BlockSpec((DL, DH * DH), lambda i: (0, 0)),
            pl.BlockSpec((1, DH * DH), lambda i: (0, 0)),
            pl.BlockSpec((1, DL), lambda i: (0, 0)),
            pl.BlockSpec((1, DL), lambda i: (0, 0)),
            pl.BlockSpec((DH, DH * DH), lambda i: (0, 0)),
            pl.BlockSpec((DH * DH, DH), lambda i: (0, 0)),
        ],
        out_specs=pl.BlockSpec((TEB, DH), lambda i: (i, 0)),
        out_shape=jax.ShapeDtypeStruct((E, DH), jnp.float32),
    )(xs, edge_attribute, el8, Wb[:DE], Wb[DE:], bb.reshape(1, DH * DH),
      centers.reshape(1, DL), beta.reshape(1, DL),
      jnp.asarray(_REP), jnp.asarray(_FOLD))


TNG = N


def _gru_body(p_ref, h_ref, bias_ref, wih_ref, whh_ref, bih_ref, bhh_ref,
              hn_ref):
    agg = p_ref[0] + p_ref[1] + bias_ref[...]
    x = jnp.maximum(agg, 0.0)
    gi = jnp.dot(x, wih_ref[...], preferred_element_type=jnp.float32) \
        + bih_ref[...]
    gh = jnp.dot(h_ref[...], whh_ref[...],
                 preferred_element_type=jnp.float32) + bhh_ref[...]
    r = jax.nn.sigmoid(gi[:, :DH] + gh[:, :DH])
    z = jax.nn.sigmoid(gi[:, DH:2 * DH] + gh[:, DH:2 * DH])
    n = jnp.tanh(gi[:, 2 * DH:] + r * gh[:, 2 * DH:])
    hn_ref[...] = (1.0 - z) * n + z * h_ref[...]


def _tc_gru(parts, h, gnn_bias, WihT, WhhT, bih, bhh):
    return pl.pallas_call(
        _gru_body,
        grid=(1,),
        in_specs=[
            pl.BlockSpec((NC, TNG, DH), lambda i: (0, 0, 0)),
            pl.BlockSpec((TNG, DH), lambda i: (0, 0)),
            pl.BlockSpec((1, DH), lambda i: (0, 0)),
            pl.BlockSpec((DH, 3 * DH), lambda i: (0, 0)),
            pl.BlockSpec((DH, 3 * DH), lambda i: (0, 0)),
            pl.BlockSpec((1, 3 * DH), lambda i: (0, 0)),
            pl.BlockSpec((1, 3 * DH), lambda i: (0, 0)),
        ],
        out_specs=pl.BlockSpec((TNG, DH), lambda i: (0, 0)),
        out_shape=jax.ShapeDtypeStruct((N, DH), jnp.float32),
    )(parts, h, gnn_bias.reshape(1, DH), WihT, WhhT,
      bih.reshape(1, 3 * DH), bhh.reshape(1, 3 * DH))


H2 = 2 * DH


def _s2s_body(x_ref, nf_ref, wih0_ref, whh0_ref, bih0_ref, bhh0_ref,
              wih1_ref, whh1_ref, bih1_ref, bhh1_ref, ws_ref, bs_ref,
              pa_ref, out_ref):
    x = x_ref[...]
    nf = nf_ref[...]
    h0 = jnp.zeros((1, H2), jnp.float32)
    c0 = jnp.zeros((1, H2), jnp.float32)
    h1 = jnp.zeros((1, H2), jnp.float32)
    c1 = jnp.zeros((1, H2), jnp.float32)
    q_star = jnp.zeros((1, 2 * H2), jnp.float32)

    def cell(xx, hh, cc, wih, whh, bih, bhh):
        g = jnp.dot(xx, wih, preferred_element_type=jnp.float32) + bih \
            + jnp.dot(hh, whh, preferred_element_type=jnp.float32) + bhh
        i = jax.nn.sigmoid(g[:, :H2])
        f = jax.nn.sigmoid(g[:, H2:2 * H2])
        gg = jnp.tanh(g[:, 2 * H2:3 * H2])
        o = jax.nn.sigmoid(g[:, 3 * H2:])
        cc2 = f * cc + i * gg
        return o * jnp.tanh(cc2), cc2

    for _ in range(PS):
        h0, c0 = cell(q_star, h0, c0, wih0_ref[...], whh0_ref[...],
                      bih0_ref[...], bhh0_ref[...])
        h1, c1 = cell(h0, h1, c1, wih1_ref[...], whh1_ref[...],
                      bih1_ref[...], bhh1_ref[...])
        q = h1
        e = lax.dot_general(x, q[:, :DH], (((1,), (1,)), ((), ())),
                            preferred_element_type=jnp.float32) \
            + lax.dot_general(nf, q[:, DH:], (((1,), (1,)), ((), ())),
                              preferred_element_type=jnp.float32)
        p = jnp.exp(e - jnp.max(e))
        alpha = p / jnp.sum(p)
        r1 = lax.dot_general(alpha, x, (((0,), (0,)), ((), ())),
                             preferred_element_type=jnp.float32)
        r2 = lax.dot_general(alpha, nf, (((0,), (0,)), ((), ())),
                             preferred_element_type=jnp.float32)
        q_star = jnp.concatenate([q, r1, r2], axis=1)

    out = jnp.dot(q_star, ws_ref[...],
                  preferred_element_type=jnp.float32) + bs_ref[...]
    out_ref[...] = jnp.maximum(out, 0.0) + pa_ref[0, 0] * jnp.minimum(out, 0.0)


def _tc_s2s(x, nf, WihT0, WhhT0, bih0, bhh0, WihT1, WhhT1, bih1, bhh1,
            Ws, bs, prelu_a):
    full = lambda shape: pl.BlockSpec(shape, lambda: tuple(0 for _ in shape))
    return pl.pallas_call(
        _s2s_body,
        grid=(),
        in_specs=[
            full((N, DH)), full((N, DH)),
            full((2 * H2, 4 * H2)), full((H2, 4 * H2)),
            full((1, 4 * H2)), full((1, 4 * H2)),
            full((H2, 4 * H2)), full((H2, 4 * H2)),
            full((1, 4 * H2)), full((1, 4 * H2)),
            full((4 * DH, DHID)), full((1, DHID)), full((1, 1)),
        ],
        out_specs=full((1, DHID)),
        out_shape=jax.ShapeDtypeStruct((1, DHID), jnp.float32),
    )(x, nf, WihT0, WhhT0, bih0.reshape(1, -1), bhh0.reshape(1, -1),
      WihT1, WhhT1, bih1.reshape(1, -1), bhh1.reshape(1, -1),
      Ws, bs.reshape(1, DHID), prelu_a.reshape(1, 1))


def kernel(node_attribute, edge_index, edge_attribute, edge_length, W1, b1,
           rbf_centers, rbf_beta, Wb, bb, gnn_bias, gru_Wih, gru_Whh,
           gru_bih, gru_bhh, lstm_Wih0, lstm_Whh0, lstm_bih0, lstm_bhh0,
           lstm_Wih1, lstm_Whh1, lstm_bih1, lstm_bhh1, Ws, bs, prelu_a):
    src = edge_index[0]
    dst3 = edge_index[1].reshape(NW, NCHUNK, CH)
    el8 = jnp.broadcast_to(edge_length.reshape(E, 1), (E, DL))
    zeros_n = jnp.zeros((N, DH), jnp.float32)
    WihT = gru_Wih.T
    WhhT = gru_Whh.T
    WihT0 = lstm_Wih0.T
    WhhT0 = lstm_Whh0.T
    WihT1 = lstm_Wih1.T
    WhhT1 = lstm_Whh1.T

    nf = _tc_nodeproj(node_attribute, W1, b1)
    x = nf
    h = nf
    for _ in range(MP):
        xs = _sc_gather(x, src)
        m = _tc_edge(xs, edge_attribute, el8, Wb, bb, rbf_centers, rbf_beta)
        parts = _sc_scatter(m, dst3, zeros_n)
        h = _tc_gru(parts, h, gnn_bias, WihT, WhhT, gru_bih, gru_bhh)
        x = h

    return _tc_s2s(x, nf, WihT0, WhhT0, lstm_bih0, lstm_bhh0,
                   WihT1, WhhT1, lstm_bih1, lstm_bhh1, Ws, bs, prelu_a)


# speedup vs baseline: 1.4294x; 1.0380x over previous
import functools

import jax
import jax.numpy as jnp
from jax import lax
from jax.experimental import pallas as pl
from jax.experimental.pallas import tpu as pltpu
from jax.experimental.pallas import tpu_sc as plsc

N = 10000
E = 160000
DN = 128
DE = 16
DL = 8
DH = 16
DHID = 4096
MP = 4
PS = 3

NC = 2
NS = 16
NW = NC * NS
EPW = E // NW
CH = 125
NCHUNK = EPW // CH
NPW = N // NS

@functools.cache
def _sc_mesh():
    return plsc.VectorSubcoreMesh(
        core_axis_name="c", subcore_axis_name="s", num_cores=NC, num_subcores=NS
    )


def _gather_body(x_hbm, src_hbm, out_hbm, idx_v, rows_v, sem):
    wid = lax.axis_index("s") * NC + lax.axis_index("c")
    base = wid * EPW
    pltpu.sync_copy(src_hbm.at[pl.ds(base, EPW)], idx_v)
    pltpu.async_copy(x_hbm.at[idx_v], rows_v, sem).wait()
    pltpu.sync_copy(rows_v, out_hbm.at[pl.ds(base, EPW)])


def _sc_gather(x, src):
    return pl.kernel(
        _gather_body,
        out_type=jax.ShapeDtypeStruct((E, DH), jnp.float32),
        mesh=_sc_mesh(),
        compiler_params=pltpu.CompilerParams(use_tc_tiling_on_sc=False),
        scratch_types=[
            pltpu.VMEM((EPW,), jnp.int32),
            pltpu.VMEM((EPW, DH), jnp.float32),
            pltpu.SemaphoreType.DMA,
        ],
    )(x, src)


def _scatter_body(m_hbm, dst_hbm, zeros_hbm, out_hbm, idx_v, rows_v, acc_sh, sem):
    c = lax.axis_index("c")
    s = lax.axis_index("s")
    wid = s * NC + c
    pltpu.sync_copy(zeros_hbm.at[pl.ds(s * NPW, NPW)],
                    acc_sh.at[pl.ds(s * NPW, NPW)])
    plsc.subcore_barrier()
    pltpu.sync_copy(m_hbm.at[pl.ds(wid * EPW, EPW)], rows_v)
    pltpu.sync_copy(dst_hbm.at[wid], idx_v)

    def chunk(j, carry):
        pltpu.sync_copy(rows_v.at[pl.ds(j * CH, CH)],
                        acc_sh.at[idx_v.at[j]], add=True)
        return carry

    lax.fori_loop(0, NCHUNK, chunk, 0)
    plsc.subcore_barrier()
    pltpu.sync_copy(acc_sh.at[pl.ds(s * NPW, NPW)],
                    out_hbm.at[c, pl.ds(s * NPW, NPW)])


def _sc_scatter(m, dst3, zeros_n):
    return pl.kernel(
        _scatter_body,
        out_type=jax.ShapeDtypeStruct((NC, N, DH), jnp.float32),
        mesh=_sc_mesh(),
        compiler_params=pltpu.CompilerParams(use_tc_tiling_on_sc=False),
        scratch_types=[
            pltpu.VMEM((NCHUNK, CH), jnp.int32),
            pltpu.VMEM((EPW, DH), jnp.float32),
            pltpu.VMEM_SHARED((N, DH), jnp.float32),
            pltpu.SemaphoreType.DMA,
        ],
    )(m, dst3, zeros_n)


TN0 = 1000


def _nodeproj_body(na_ref, w1_ref, b1_ref, out_ref):
    out_ref[...] = jnp.maximum(
        jnp.dot(na_ref[...], w1_ref[...],
                preferred_element_type=jnp.float32) + b1_ref[...], 0.0)


def _tc_nodeproj(node_attribute, W1, b1):
    return pl.pallas_call(
        _nodeproj_body,
        grid=(N // TN0,),
        in_specs=[
            pl.BlockSpec((TN0, DN), lambda i: (i, 0)),
            pl.BlockSpec((DN, DH), lambda i: (0, 0)),
            pl.BlockSpec((1, DH), lambda i: (0, 0)),
        ],
        out_specs=pl.BlockSpec((TN0, DH), lambda i: (i, 0)),
        out_shape=jax.ShapeDtypeStruct((N, DH), jnp.float32),
    )(node_attribute, W1, b1.reshape(1, DH))


TEB = 6400


import numpy as _np

_REP = _np.zeros((DH, DH * DH), _np.float32)
_FOLD = _np.zeros((DH * DH, DH), _np.float32)
for _i in range(DH):
    for _o in range(DH):
        _REP[_i, DH * _i + _o] = 1.0
        _FOLD[DH * _i + _o, _o] = 1.0


SUB = 640
NSUB = TEB // SUB


def _edge_body(xs_ref, ea_ref, el_ref, wba_ref, wbb_ref, bb_ref, cen_ref,
               beta_ref, rep_ref, fold_ref, m_ref):
    for sub in range(NSUB):
        sl = pl.ds(sub * SUB, SUB)
        diff = el_ref[sl, :] - cen_ref[...]
        rbf = jnp.exp(-beta_ref[...] * diff * diff)
        wf = jnp.dot(ea_ref[sl, :], wba_ref[...],
                     preferred_element_type=jnp.float32) \
            + jnp.dot(rbf, wbb_ref[...],
                      preferred_element_type=jnp.float32) + bb_ref[...]
        xr = jnp.dot(xs_ref[sl, :], rep_ref[...],
                     preferred_element_type=jnp.float32)
        m_ref[sl, :] = jnp.dot(wf * xr, fold_ref[...],
                               preferred_element_type=jnp.float32)


def _tc_edge(xs, edge_attribute, el8, Wb, bb, centers, beta):
    return pl.pallas_call(
        _edge_body,
        grid=(E // TEB,),
        in_specs=[
            pl.BlockSpec((TEB, DH), lambda i: (i, 0)),
            pl.BlockSpec((TEB, DE), lambda i: (i, 0)),
            pl.BlockSpec((TEB, DL), lambda i: (i, 0)),
            pl.BlockSpec((DE, DH * DH), lambda i: (0, 0)),
            pl.BlockSpec((DL, DH * DH), lambda i: (0, 0)),
            pl.BlockSpec((1, DH * DH), lambda i: (0, 0)),
            pl.BlockSpec((1, DL), lambda i: (0, 0)),
            pl.BlockSpec((1, DL), lambda i: (0, 0)),
            pl.BlockSpec((DH, DH * DH), lambda i: (0, 0)),
            pl.BlockSpec((DH * DH, DH), lambda i: (0, 0)),
        ],
        out_specs=pl.BlockSpec((TEB, DH), lambda i: (i, 0)),
        out_shape=jax.ShapeDtypeStruct((E, DH), jnp.float32),
    )(xs, edge_attribute, el8, Wb[:DE], Wb[DE:], bb.reshape(1, DH * DH),
      centers.reshape(1, DL), beta.reshape(1, DL),
      jnp.asarray(_REP), jnp.asarray(_FOLD))


TNG = N


def _gru_body(p_ref, h_ref, bias_ref, wih_ref, whh_ref, bih_ref, bhh_ref,
              hn_ref):
    agg = p_ref[0] + p_ref[1] + bias_ref[...]
    x = jnp.maximum(agg, 0.0)
    gi = jnp.dot(x, wih_ref[...], preferred_element_type=jnp.float32) \
        + bih_ref[...]
    gh = jnp.dot(h_ref[...], whh_ref[...],
                 preferred_element_type=jnp.float32) + bhh_ref[...]
    r = jax.nn.sigmoid(gi[:, :DH] + gh[:, :DH])
    z = jax.nn.sigmoid(gi[:, DH:2 * DH] + gh[:, DH:2 * DH])
    n = jnp.tanh(gi[:, 2 * DH:] + r * gh[:, 2 * DH:])
    hn_ref[...] = (1.0 - z) * n + z * h_ref[...]


def _tc_gru(parts, h, gnn_bias, WihT, WhhT, bih, bhh):
    return pl.pallas_call(
        _gru_body,
        grid=(1,),
        in_specs=[
            pl.BlockSpec((NC, TNG, DH), lambda i: (0, 0, 0)),
            pl.BlockSpec((TNG, DH), lambda i: (0, 0)),
            pl.BlockSpec((1, DH), lambda i: (0, 0)),
            pl.BlockSpec((DH, 3 * DH), lambda i: (0, 0)),
            pl.BlockSpec((DH, 3 * DH), lambda i: (0, 0)),
            pl.BlockSpec((1, 3 * DH), lambda i: (0, 0)),
            pl.BlockSpec((1, 3 * DH), lambda i: (0, 0)),
        ],
        out_specs=pl.BlockSpec((TNG, DH), lambda i: (0, 0)),
        out_shape=jax.ShapeDtypeStruct((N, DH), jnp.float32),
    )(parts, h, gnn_bias.reshape(1, DH), WihT, WhhT,
      bih.reshape(1, 3 * DH), bhh.reshape(1, 3 * DH))


H2 = 2 * DH


def _s2s_body(x_ref, nf_ref, wih0_ref, whh0_ref, bih0_ref, bhh0_ref,
              wih1_ref, whh1_ref, bih1_ref, bhh1_ref, ws_ref, bs_ref,
              pa_ref, out_ref):
    x = x_ref[...]
    nf = nf_ref[...]
    h0 = jnp.zeros((1, H2), jnp.float32)
    c0 = jnp.zeros((1, H2), jnp.float32)
    h1 = jnp.zeros((1, H2), jnp.float32)
    c1 = jnp.zeros((1, H2), jnp.float32)
    q_star = jnp.zeros((1, 2 * H2), jnp.float32)

    def cell(xx, hh, cc, wih, whh, bih, bhh):
        g = jnp.dot(xx, wih, preferred_element_type=jnp.float32) + bih \
            + jnp.dot(hh, whh, preferred_element_type=jnp.float32) + bhh
        i = jax.nn.sigmoid(g[:, :H2])
        f = jax.nn.sigmoid(g[:, H2:2 * H2])
        gg = jnp.tanh(g[:, 2 * H2:3 * H2])
        o = jax.nn.sigmoid(g[:, 3 * H2:])
        cc2 = f * cc + i * gg
        return o * jnp.tanh(cc2), cc2

    for _ in range(PS):
        h0, c0 = cell(q_star, h0, c0, wih0_ref[...], whh0_ref[...],
                      bih0_ref[...], bhh0_ref[...])
        h1, c1 = cell(h0, h1, c1, wih1_ref[...], whh1_ref[...],
                      bih1_ref[...], bhh1_ref[...])
        q = h1
        e = lax.dot_general(x, q[:, :DH], (((1,), (1,)), ((), ())),
                            preferred_element_type=jnp.float32) \
            + lax.dot_general(nf, q[:, DH:], (((1,), (1,)), ((), ())),
                              preferred_element_type=jnp.float32)
        p = jnp.exp(e - jnp.max(e))
        alpha = p / jnp.sum(p)
        r1 = lax.dot_general(alpha, x, (((0,), (0,)), ((), ())),
                             preferred_element_type=jnp.float32)
        r2 = lax.dot_general(alpha, nf, (((0,), (0,)), ((), ())),
                             preferred_element_type=jnp.float32)
        q_star = jnp.concatenate([q, r1, r2], axis=1)

    out = jnp.dot(q_star, ws_ref[...],
                  preferred_element_type=jnp.float32) + bs_ref[...]
    out_ref[...] = jnp.maximum(out, 0.0) + pa_ref[0, 0] * jnp.minimum(out, 0.0)


def _tc_s2s(x, nf, WihT0, WhhT0, bih0, bhh0, WihT1, WhhT1, bih1, bhh1,
            Ws, bs, prelu_a):
    full = lambda shape: pl.BlockSpec(shape, lambda: tuple(0 for _ in shape))
    return pl.pallas_call(
        _s2s_body,
        grid=(),
        in_specs=[
            full((N, DH)), full((N, DH)),
            full((2 * H2, 4 * H2)), full((H2, 4 * H2)),
            full((1, 4 * H2)), full((1, 4 * H2)),
            full((H2, 4 * H2)), full((H2, 4 * H2)),
            full((1, 4 * H2)), full((1, 4 * H2)),
            full((4 * DH, DHID)), full((1, DHID)), full((1, 1)),
        ],
        out_specs=full((1, DHID)),
        out_shape=jax.ShapeDtypeStruct((1, DHID), jnp.float32),
    )(x, nf, WihT0, WhhT0, bih0.reshape(1, -1), bhh0.reshape(1, -1),
      WihT1, WhhT1, bih1.reshape(1, -1), bhh1.reshape(1, -1),
      Ws, bs.reshape(1, DHID), prelu_a.reshape(1, 1))


def kernel(node_attribute, edge_index, edge_attribute, edge_length, W1, b1,
           rbf_centers, rbf_beta, Wb, bb, gnn_bias, gru_Wih, gru_Whh,
           gru_bih, gru_bhh, lstm_Wih0, lstm_Whh0, lstm_bih0, lstm_bhh0,
           lstm_Wih1, lstm_Whh1, lstm_bih1, lstm_bhh1, Ws, bs, prelu_a):
    src = edge_index[0]
    dst3 = edge_index[1].reshape(NW, NCHUNK, CH)
    el8 = jnp.broadcast_to(edge_length.reshape(E, 1), (E, DL))
    zeros_n = jnp.zeros((N, DH), jnp.float32)
    WihT = gru_Wih.T
    WhhT = gru_Whh.T
    WihT0 = lstm_Wih0.T
    WhhT0 = lstm_Whh0.T
    WihT1 = lstm_Wih1.T
    WhhT1 = lstm_Whh1.T

    nf = _tc_nodeproj(node_attribute, W1, b1)
    x = nf
    h = nf
    for _ in range(MP):
        xs = _sc_gather(x, src)
        m = _tc_edge(xs, edge_attribute, el8, Wb, bb, rbf_centers, rbf_beta)
        parts = _sc_scatter(m, dst3, zeros_n)
        h = _tc_gru(parts, h, gnn_bias, WihT, WhhT, gru_bih, gru_bhh)
        x = h

    return _tc_s2s(x, nf, WihT0, WhhT0, lstm_bih0, lstm_bhh0,
                   WihT1, WhhT1, lstm_bih1, lstm_bhh1, Ws, bs, prelu_a)
